# Initial kernel scaffold; baseline (speedup 1.0000x reference)
import jax
import jax.numpy as jnp
from jax.experimental import pallas as pl


def kernel(x, edge_index, pos_encoding, W, b, W_pos, b_pos):
    raise NotImplementedError("write your pallas kernel here")


# speedup vs baseline: 30.4231x; 30.4231x over previous
import functools

import jax
import jax.numpy as jnp
from jax import lax
from jax.experimental import pallas as pl
from jax.experimental.pallas import tpu as pltpu
from jax.experimental.pallas import tpu_sc as plsc

N_NODES = 10000
N_EDGES = 320000
CH = 128

NW = 32
E_PAD = 327680
EW_ROWS = 80
CHUNK = 128
EW_CHUNKS = 80
N_PAD = 10240
ROWS_PER_TILE = N_PAD // 16

_mesh = plsc.VectorSubcoreMesh(core_axis_name="c", subcore_axis_name="s")


@functools.partial(
    pl.kernel,
    out_type=jax.ShapeDtypeStruct((2, N_PAD), jnp.float32),
    mesh=_mesh,
    scratch_types=[
        pltpu.VMEM((EW_ROWS, 128), jnp.int32),
        pltpu.VMEM((128,), jnp.float32),
        pltpu.VMEM((ROWS_PER_TILE,), jnp.float32),
        pltpu.VMEM_SHARED((N_PAD,), jnp.float32),
    ],
)
def _sc_degree(dst_hbm, deg_out, idx_v, ones_v, stage_v, shared_deg):
    c = lax.axis_index("c")
    s = lax.axis_index("s")
    wid = c * 16 + s

    for i in range(128 // 16):
        ones_v[pl.ds(i * 16, 16)] = jnp.ones((16,), jnp.float32)
    for i in range(ROWS_PER_TILE // 16):
        stage_v[pl.ds(i * 16, 16)] = jnp.zeros((16,), jnp.float32)
    pltpu.sync_copy(stage_v, shared_deg.at[pl.ds(s * ROWS_PER_TILE, ROWS_PER_TILE)])
    pltpu.sync_copy(dst_hbm.at[pl.ds(wid * EW_ROWS, EW_ROWS)], idx_v)
    plsc.subcore_barrier()

    def body(j, carry):
        pltpu.sync_copy(ones_v, shared_deg.at[idx_v.at[j]], add=True)
        return carry

    lax.fori_loop(0, EW_ROWS, body, 0)
    plsc.subcore_barrier()

    pltpu.sync_copy(shared_deg.at[pl.ds(s * ROWS_PER_TILE, ROWS_PER_TILE)], stage_v)
    pltpu.sync_copy(stage_v, deg_out.at[c].at[pl.ds(s * ROWS_PER_TILE, ROWS_PER_TILE)])


@functools.partial(
    pl.kernel,
    out_type=jax.ShapeDtypeStruct((2, N_PAD, CH), jnp.float32),
    mesh=_mesh,
    scratch_types=[
        pltpu.VMEM((EW_CHUNKS, CHUNK), jnp.int32),
        pltpu.VMEM((EW_CHUNKS, CHUNK), jnp.int32),
        pltpu.VMEM((CHUNK, CH), jnp.float32),
        pltpu.VMEM_SHARED((N_PAD, CH), jnp.float32),
        pltpu.SemaphoreType.DMA,
    ],
)
def _sc_aggregate(g_hbm, src_hbm, dst_hbm, acc_out,
                  src_v, dst_v, rows_a, shared_acc, sem_a):
    c = lax.axis_index("c")
    s = lax.axis_index("s")
    wid = c * 16 + s

    def zero_body(i, carry):
        for k in range(CH // 16):
            rows_a[i, pl.ds(k * 16, 16)] = jnp.zeros((16,), jnp.float32)
        return carry

    lax.fori_loop(0, CHUNK, zero_body, 0)
    for m in range(ROWS_PER_TILE // CHUNK):
        pltpu.sync_copy(rows_a,
                        shared_acc.at[pl.ds(s * ROWS_PER_TILE + m * CHUNK, CHUNK)])
    pltpu.sync_copy(src_hbm.at[pl.ds(wid * EW_CHUNKS, EW_CHUNKS)], src_v)
    pltpu.sync_copy(dst_hbm.at[pl.ds(wid * EW_CHUNKS, EW_CHUNKS)], dst_v)
    plsc.subcore_barrier()

    def body(j, carry):
        pltpu.async_copy(g_hbm.at[src_v.at[j]], rows_a, sem_a).wait()
        pltpu.sync_copy(rows_a, shared_acc.at[dst_v.at[j]], add=True)
        return carry

    lax.fori_loop(0, EW_CHUNKS, body, 0)
    plsc.subcore_barrier()

    for m in range(ROWS_PER_TILE // CHUNK):
        sl = pl.ds(s * ROWS_PER_TILE + m * CHUNK, CHUNK)
        pltpu.sync_copy(shared_acc.at[sl], rows_a)
        pltpu.sync_copy(rows_a, acc_out.at[c].at[sl])


def _tc_prep(x_pad, W, deg_col2):
    def body(x_ref, w_ref, deg_ref, g_ref, dfull_ref):
        deg = deg_ref[0] + deg_ref[1]
        dinv = lax.rsqrt(deg + 1.0)
        dfull = jnp.broadcast_to(dinv, (N_PAD, CH))
        h = jnp.dot(x_ref[...], w_ref[...], preferred_element_type=jnp.float32)
        g_ref[...] = h * dfull
        dfull_ref[...] = dfull

    return pl.pallas_call(
        body,
        out_shape=(
            jax.ShapeDtypeStruct((N_PAD, CH), jnp.float32),
            jax.ShapeDtypeStruct((N_PAD, CH), jnp.float32),
        ),
    )(x_pad, W, deg_col2)


def _tc_combine(acc2, g, dfull, pos, W_pos, b_row, b_pos_row):
    def body(acc_ref, g_ref, dfull_ref, pos_ref, wp_ref, b_ref, bp_ref, out_ref):
        acc = acc_ref[0] + acc_ref[1] + g_ref[...]
        pos_lin = jnp.dot(pos_ref[...], wp_ref[...],
                          preferred_element_type=jnp.float32)
        out_ref[...] = dfull_ref[...] * acc + pos_lin + b_ref[...] + bp_ref[...]

    return pl.pallas_call(
        body,
        out_shape=jax.ShapeDtypeStruct((N_NODES, CH), jnp.float32),
    )(acc2, g, dfull, pos, W_pos, b_row, b_pos_row)


def kernel(x, edge_index, pos_encoding, W, b, W_pos, b_pos):
    src = edge_index[0].astype(jnp.int32)
    dst = edge_index[1].astype(jnp.int32)

    n_extra = E_PAD - N_EDGES
    pad_iota = lax.iota(jnp.int32, n_extra)
    src_pad = jnp.concatenate([src, pad_iota % N_NODES])
    dst_pad = jnp.concatenate([dst, N_NODES + pad_iota % (N_PAD - N_NODES)])
    src2d = src_pad.reshape(E_PAD // CHUNK, CHUNK)
    dst2d = dst_pad.reshape(E_PAD // CHUNK, CHUNK)
    deg2 = _sc_degree(dst2d)

    x_pad = jnp.pad(x, ((0, N_PAD - N_NODES), (0, 0)))
    g, dfull = _tc_prep(x_pad, W, deg2.reshape(2, N_PAD, 1))

    acc2 = _sc_aggregate(g, src2d, dst2d)

    return _tc_combine(
        acc2[:, :N_NODES],
        g[:N_NODES],
        dfull[:N_NODES],
        pos_encoding,
        W_pos,
        b.reshape(1, CH),
        b_pos.reshape(1, CH),
    )


# speedup vs baseline: 41.0706x; 1.3500x over previous
import functools

import jax
import jax.numpy as jnp
from jax import lax
from jax.experimental import pallas as pl
from jax.experimental.pallas import tpu as pltpu
from jax.experimental.pallas import tpu_sc as plsc

N_NODES = 10000
N_EDGES = 320000
CH = 128

NW = 32
E_PAD = 327680
EW_ROWS = 80
CHUNK = 128
EW_CHUNKS = 80
IDX_GROUPS = 2
GCHUNKS = EW_CHUNKS // IDX_GROUPS
N_PAD = 10240
ROWS_PER_TILE = N_PAD // 16

_mesh = plsc.VectorSubcoreMesh(core_axis_name="c", subcore_axis_name="s")


@functools.partial(
    pl.kernel,
    out_type=jax.ShapeDtypeStruct((2, N_PAD), jnp.float32),
    mesh=_mesh,
    scratch_types=[
        pltpu.VMEM((EW_ROWS, 128), jnp.int32),
        pltpu.VMEM((128,), jnp.float32),
        pltpu.VMEM((ROWS_PER_TILE,), jnp.float32),
        pltpu.VMEM_SHARED((N_PAD,), jnp.float32),
    ],
)
def _sc_degree(dst_hbm, deg_out, idx_v, ones_v, stage_v, shared_deg):
    c = lax.axis_index("c")
    s = lax.axis_index("s")
    wid = c * 16 + s

    for i in range(128 // 16):
        ones_v[pl.ds(i * 16, 16)] = jnp.ones((16,), jnp.float32)
    for i in range(ROWS_PER_TILE // 16):
        stage_v[pl.ds(i * 16, 16)] = jnp.zeros((16,), jnp.float32)
    pltpu.sync_copy(stage_v, shared_deg.at[pl.ds(s * ROWS_PER_TILE, ROWS_PER_TILE)])
    pltpu.sync_copy(dst_hbm.at[pl.ds(wid * EW_ROWS, EW_ROWS)], idx_v)
    plsc.subcore_barrier()

    def body(j, carry):
        pltpu.sync_copy(ones_v, shared_deg.at[idx_v.at[j]], add=True)
        return carry

    lax.fori_loop(0, EW_ROWS, body, 0)
    plsc.subcore_barrier()

    pltpu.sync_copy(shared_deg.at[pl.ds(s * ROWS_PER_TILE, ROWS_PER_TILE)], stage_v)
    pltpu.sync_copy(stage_v, deg_out.at[c].at[pl.ds(s * ROWS_PER_TILE, ROWS_PER_TILE)])


@functools.partial(
    pl.kernel,
    out_type=jax.ShapeDtypeStruct((2, N_PAD, CH), jnp.float32),
    mesh=_mesh,
    scratch_types=[
        pltpu.VMEM((GCHUNKS, CHUNK), jnp.int32),
        pltpu.VMEM((GCHUNKS, CHUNK), jnp.int32),
        pltpu.VMEM((CHUNK, CH), jnp.float32),
        pltpu.VMEM((CHUNK, CH), jnp.float32),
        pltpu.VMEM_SHARED((N_PAD, CH), jnp.float32),
        pltpu.SemaphoreType.DMA,
        pltpu.SemaphoreType.DMA,
    ],
)
def _sc_aggregate(g_hbm, src_hbm, dst_hbm, acc_out,
                  src_v, dst_v, rows_a, rows_b, shared_acc, sem_a, sem_b):
    c = lax.axis_index("c")
    s = lax.axis_index("s")
    wid = c * 16 + s

    def zero_body(i, carry):
        for k in range(CH // 16):
            rows_a[i, pl.ds(k * 16, 16)] = jnp.zeros((16,), jnp.float32)
        return carry

    lax.fori_loop(0, CHUNK, zero_body, 0)
    for m in range(ROWS_PER_TILE // CHUNK):
        pltpu.sync_copy(rows_a,
                        shared_acc.at[pl.ds(s * ROWS_PER_TILE + m * CHUNK, CHUNK)])
    plsc.subcore_barrier()

    for grp in range(IDX_GROUPS):
        base = wid * EW_CHUNKS + grp * GCHUNKS
        pltpu.sync_copy(src_hbm.at[pl.ds(base, GCHUNKS)], src_v)
        pltpu.sync_copy(dst_hbm.at[pl.ds(base, GCHUNKS)], dst_v)
        pltpu.async_copy(g_hbm.at[src_v.at[0]], rows_a, sem_a)

        def body(j, carry):
            @pl.when(j % 2 == 0)
            def _():
                @pl.when(j + 1 < GCHUNKS)
                def _():
                    pltpu.async_copy(g_hbm.at[src_v.at[j + 1]], rows_b, sem_b)
                pltpu.make_async_copy(g_hbm.at[src_v.at[j]], rows_a, sem_a).wait()
                pltpu.sync_copy(rows_a, shared_acc.at[dst_v.at[j]], add=True)

            @pl.when(j % 2 == 1)
            def _():
                @pl.when(j + 1 < GCHUNKS)
                def _():
                    pltpu.async_copy(g_hbm.at[src_v.at[j + 1]], rows_a, sem_a)
                pltpu.make_async_copy(g_hbm.at[src_v.at[j]], rows_b, sem_b).wait()
                pltpu.sync_copy(rows_b, shared_acc.at[dst_v.at[j]], add=True)

            return carry

        lax.fori_loop(0, GCHUNKS, body, 0)
    plsc.subcore_barrier()

    for m in range(ROWS_PER_TILE // CHUNK):
        sl = pl.ds(s * ROWS_PER_TILE + m * CHUNK, CHUNK)
        pltpu.sync_copy(shared_acc.at[sl], rows_a)
        pltpu.sync_copy(rows_a, acc_out.at[c].at[sl])


def _tc_prep(x_pad, W, deg_col2):
    def body(x_ref, w_ref, deg_ref, g_ref, dfull_ref):
        deg = deg_ref[0] + deg_ref[1]
        dinv = lax.rsqrt(deg + 1.0)
        dfull = jnp.broadcast_to(dinv, (N_PAD, CH))
        h = jnp.dot(x_ref[...], w_ref[...], preferred_element_type=jnp.float32)
        g_ref[...] = h * dfull
        dfull_ref[...] = dfull

    return pl.pallas_call(
        body,
        out_shape=(
            jax.ShapeDtypeStruct((N_PAD, CH), jnp.float32),
            jax.ShapeDtypeStruct((N_PAD, CH), jnp.float32),
        ),
    )(x_pad, W, deg_col2)


def _tc_combine(acc2, g, dfull, pos, W_pos, b_row, b_pos_row):
    def body(acc_ref, g_ref, dfull_ref, pos_ref, wp_ref, b_ref, bp_ref, out_ref):
        acc = acc_ref[0] + acc_ref[1] + g_ref[...]
        pos_lin = jnp.dot(pos_ref[...], wp_ref[...],
                          preferred_element_type=jnp.float32)
        out_ref[...] = dfull_ref[...] * acc + pos_lin + b_ref[...] + bp_ref[...]

    return pl.pallas_call(
        body,
        out_shape=jax.ShapeDtypeStruct((N_NODES, CH), jnp.float32),
    )(acc2, g, dfull, pos, W_pos, b_row, b_pos_row)


def kernel(x, edge_index, pos_encoding, W, b, W_pos, b_pos):
    src = edge_index[0].astype(jnp.int32)
    dst = edge_index[1].astype(jnp.int32)

    n_extra = E_PAD - N_EDGES
    pad_iota = lax.iota(jnp.int32, n_extra)
    src_pad = jnp.concatenate([src, pad_iota % N_NODES])
    dst_pad = jnp.concatenate([dst, N_NODES + pad_iota % (N_PAD - N_NODES)])
    src2d = src_pad.reshape(E_PAD // CHUNK, CHUNK)
    dst2d = dst_pad.reshape(E_PAD // CHUNK, CHUNK)
    deg2 = _sc_degree(dst2d)

    x_pad = jnp.pad(x, ((0, N_PAD - N_NODES), (0, 0)))
    g, dfull = _tc_prep(x_pad, W, deg2.reshape(2, N_PAD, 1))

    acc2 = _sc_aggregate(g, src2d, dst2d)

    return _tc_combine(
        acc2[:, :N_NODES],
        g[:N_NODES],
        dfull[:N_NODES],
        pos_encoding,
        W_pos,
        b.reshape(1, CH),
        b_pos.reshape(1, CH),
    )


# speedup vs baseline: 42.4928x; 1.0346x over previous
import functools

import jax
import jax.numpy as jnp
from jax import lax
from jax.experimental import pallas as pl
from jax.experimental.pallas import tpu as pltpu
from jax.experimental.pallas import tpu_sc as plsc

N_NODES = 10000
N_EDGES = 320000
CH = 128

NW = 32
E_PAD = 327680
EW_ROWS = 80
CHUNK = 128
EW_CHUNKS = 80
IDX_GROUPS = 2
GCHUNKS = EW_CHUNKS // IDX_GROUPS
N_PAD = 10240
ROWS_PER_TILE = N_PAD // 16

_mesh = plsc.VectorSubcoreMesh(core_axis_name="c", subcore_axis_name="s")


@functools.partial(
    pl.kernel,
    out_type=jax.ShapeDtypeStruct((2, N_PAD), jnp.float32),
    mesh=_mesh,
    scratch_types=[
        pltpu.VMEM((EW_ROWS, 128), jnp.int32),
        pltpu.VMEM((128,), jnp.float32),
        pltpu.VMEM((ROWS_PER_TILE,), jnp.float32),
        pltpu.VMEM_SHARED((N_PAD,), jnp.float32),
    ],
)
def _sc_degree(dst_hbm, deg_out, idx_v, ones_v, stage_v, shared_deg):
    c = lax.axis_index("c")
    s = lax.axis_index("s")
    wid = c * 16 + s

    for i in range(128 // 16):
        ones_v[pl.ds(i * 16, 16)] = jnp.ones((16,), jnp.float32)
    for i in range(ROWS_PER_TILE // 16):
        stage_v[pl.ds(i * 16, 16)] = jnp.zeros((16,), jnp.float32)
    pltpu.sync_copy(stage_v, shared_deg.at[pl.ds(s * ROWS_PER_TILE, ROWS_PER_TILE)])
    pltpu.sync_copy(dst_hbm.at[pl.ds(wid * EW_ROWS, EW_ROWS)], idx_v)
    plsc.subcore_barrier()

    def body(j, carry):
        pltpu.sync_copy(ones_v, shared_deg.at[idx_v.at[j]], add=True)
        return carry

    lax.fori_loop(0, EW_ROWS, body, 0)
    plsc.subcore_barrier()

    pltpu.sync_copy(shared_deg.at[pl.ds(s * ROWS_PER_TILE, ROWS_PER_TILE)], stage_v)
    pltpu.sync_copy(stage_v, deg_out.at[c].at[pl.ds(s * ROWS_PER_TILE, ROWS_PER_TILE)])


@functools.partial(
    pl.kernel,
    out_type=jax.ShapeDtypeStruct((2, N_PAD, CH), jnp.float32),
    mesh=_mesh,
    scratch_types=[
        pltpu.VMEM((GCHUNKS, CHUNK), jnp.int32),
        pltpu.VMEM((GCHUNKS, CHUNK), jnp.int32),
        pltpu.VMEM((CHUNK, CH), jnp.float32),
        pltpu.VMEM((CHUNK, CH), jnp.float32),
        pltpu.VMEM_SHARED((N_PAD, CH), jnp.float32),
        pltpu.SemaphoreType.DMA,
        pltpu.SemaphoreType.DMA,
    ],
)
def _sc_aggregate(g_hbm, src_hbm, dst_hbm, zeros_hbm, acc_out,
                  src_v, dst_v, rows_a, rows_b, shared_acc, sem_a, sem_b):
    c = lax.axis_index("c")
    s = lax.axis_index("s")
    wid = c * 16 + s

    pltpu.sync_copy(zeros_hbm, rows_a)
    for m in range(ROWS_PER_TILE // CHUNK):
        pltpu.sync_copy(rows_a,
                        shared_acc.at[pl.ds(s * ROWS_PER_TILE + m * CHUNK, CHUNK)])
    plsc.subcore_barrier()

    for grp in range(IDX_GROUPS):
        base = wid * EW_CHUNKS + grp * GCHUNKS
        pltpu.sync_copy(src_hbm.at[pl.ds(base, GCHUNKS)], src_v)
        pltpu.sync_copy(dst_hbm.at[pl.ds(base, GCHUNKS)], dst_v)
        pltpu.async_copy(g_hbm.at[src_v.at[0]], rows_a, sem_a)

        def body(j, carry):
            @pl.when(j % 2 == 0)
            def _():
                @pl.when(j + 1 < GCHUNKS)
                def _():
                    pltpu.async_copy(g_hbm.at[src_v.at[j + 1]], rows_b, sem_b)
                pltpu.make_async_copy(g_hbm.at[src_v.at[j]], rows_a, sem_a).wait()
                pltpu.sync_copy(rows_a, shared_acc.at[dst_v.at[j]], add=True)

            @pl.when(j % 2 == 1)
            def _():
                @pl.when(j + 1 < GCHUNKS)
                def _():
                    pltpu.async_copy(g_hbm.at[src_v.at[j + 1]], rows_a, sem_a)
                pltpu.make_async_copy(g_hbm.at[src_v.at[j]], rows_b, sem_b).wait()
                pltpu.sync_copy(rows_b, shared_acc.at[dst_v.at[j]], add=True)

            return carry

        lax.fori_loop(0, GCHUNKS, body, 0)
    plsc.subcore_barrier()

    out_sls = [pl.ds(s * ROWS_PER_TILE + m * CHUNK, CHUNK)
               for m in range(ROWS_PER_TILE // CHUNK)]
    bufs = [rows_a, rows_b]
    sems = [sem_a, sem_b]
    for m, sl in enumerate(out_sls):
        buf, sem = bufs[m % 2], sems[m % 2]
        if m >= 2:
            pltpu.make_async_copy(buf, acc_out.at[c].at[out_sls[m - 2]], sem).wait()
        pltpu.sync_copy(shared_acc.at[sl], buf)
        pltpu.async_copy(buf, acc_out.at[c].at[sl], sem)
    for m in (len(out_sls) - 2, len(out_sls) - 1):
        pltpu.make_async_copy(bufs[m % 2], acc_out.at[c].at[out_sls[m]],
                              sems[m % 2]).wait()


def _tc_prep(x, W, deg_col2):
    def body(x_ref, w_ref, deg_ref, g_ref, dfull_ref):
        deg = deg_ref[0, 0:N_NODES] + deg_ref[1, 0:N_NODES]
        dinv = lax.rsqrt(deg + 1.0)
        dfull = jnp.broadcast_to(dinv, (N_NODES, CH))
        h = jnp.dot(x_ref[...], w_ref[...], preferred_element_type=jnp.float32)
        g_ref[...] = h * dfull
        dfull_ref[...] = dfull

    return pl.pallas_call(
        body,
        out_shape=(
            jax.ShapeDtypeStruct((N_NODES, CH), jnp.float32),
            jax.ShapeDtypeStruct((N_NODES, CH), jnp.float32),
        ),
    )(x, W, deg_col2)


def _tc_combine(acc2, g, dfull, pos, W_pos, b_row, b_pos_row):
    def body(acc_ref, g_ref, dfull_ref, pos_ref, wp_ref, b_ref, bp_ref, out_ref):
        acc = acc_ref[0, 0:N_NODES] + acc_ref[1, 0:N_NODES] + g_ref[...]
        pos_lin = jnp.dot(pos_ref[...], wp_ref[...],
                          preferred_element_type=jnp.float32)
        out_ref[...] = dfull_ref[...] * acc + pos_lin + b_ref[...] + bp_ref[...]

    return pl.pallas_call(
        body,
        out_shape=jax.ShapeDtypeStruct((N_NODES, CH), jnp.float32),
    )(acc2, g, dfull, pos, W_pos, b_row, b_pos_row)


def kernel(x, edge_index, pos_encoding, W, b, W_pos, b_pos):
    src = edge_index[0].astype(jnp.int32)
    dst = edge_index[1].astype(jnp.int32)

    n_extra = E_PAD - N_EDGES
    pad_iota = lax.iota(jnp.int32, n_extra)
    src_pad = jnp.concatenate([src, pad_iota % N_NODES])
    dst_pad = jnp.concatenate([dst, N_NODES + pad_iota % (N_PAD - N_NODES)])
    src2d = src_pad.reshape(E_PAD // CHUNK, CHUNK)
    dst2d = dst_pad.reshape(E_PAD // CHUNK, CHUNK)
    deg2 = _sc_degree(dst2d)

    g, dfull = _tc_prep(x, W, deg2.reshape(2, N_PAD, 1))

    acc2 = _sc_aggregate(g, src2d, dst2d, jnp.zeros((CHUNK, CH), jnp.float32))

    return _tc_combine(
        acc2,
        g,
        dfull,
        pos_encoding,
        W_pos,
        b.reshape(1, CH),
        b_pos.reshape(1, CH),
    )


# speedup vs baseline: 44.1819x; 1.0398x over previous
import functools

import jax
import jax.numpy as jnp
from jax import lax
from jax.experimental import pallas as pl
from jax.experimental.pallas import tpu as pltpu
from jax.experimental.pallas import tpu_sc as plsc

N_NODES = 10000
N_EDGES = 320000
CH = 128

NW = 32
E_PAD = 327680
EROWS = E_PAD // 128
EW_ROWS = 80
CHUNK = 128
EW_CHUNKS = 80
IDX_GROUPS = 2
GCHUNKS = EW_CHUNKS // IDX_GROUPS
N_PAD = 10240
ROWS_PER_TILE = N_PAD // 16

_mesh = plsc.VectorSubcoreMesh(core_axis_name="c", subcore_axis_name="s")


@functools.partial(
    pl.kernel,
    out_type=jax.ShapeDtypeStruct((2, N_PAD), jnp.float32),
    mesh=_mesh,
    scratch_types=[
        pltpu.VMEM((EW_ROWS, 128), jnp.int32),
        pltpu.VMEM((128,), jnp.float32),
        pltpu.VMEM((ROWS_PER_TILE,), jnp.float32),
        pltpu.VMEM_SHARED((N_PAD,), jnp.float32),
    ],
)
def _sc_degree(edges_hbm, deg_out, idx_v, ones_v, stage_v, shared_deg):
    c = lax.axis_index("c")
    s = lax.axis_index("s")
    wid = c * 16 + s

    for i in range(128 // 16):
        ones_v[pl.ds(i * 16, 16)] = jnp.ones((16,), jnp.float32)
    for i in range(ROWS_PER_TILE // 16):
        stage_v[pl.ds(i * 16, 16)] = jnp.zeros((16,), jnp.float32)
    pltpu.sync_copy(stage_v, shared_deg.at[pl.ds(s * ROWS_PER_TILE, ROWS_PER_TILE)])
    pltpu.sync_copy(edges_hbm.at[pl.ds(EROWS + wid * EW_ROWS, EW_ROWS)], idx_v)
    plsc.subcore_barrier()

    def body(j, carry):
        pltpu.sync_copy(ones_v, shared_deg.at[idx_v.at[j]], add=True)
        return carry

    lax.fori_loop(0, EW_ROWS, body, 0)
    plsc.subcore_barrier()

    pltpu.sync_copy(shared_deg.at[pl.ds(s * ROWS_PER_TILE, ROWS_PER_TILE)], stage_v)
    pltpu.sync_copy(stage_v, deg_out.at[c].at[pl.ds(s * ROWS_PER_TILE, ROWS_PER_TILE)])


@functools.partial(
    pl.kernel,
    out_type=jax.ShapeDtypeStruct((2, N_PAD, CH), jnp.float32),
    mesh=_mesh,
    scratch_types=[
        pltpu.VMEM((GCHUNKS, CHUNK), jnp.int32),
        pltpu.VMEM((GCHUNKS, CHUNK), jnp.int32),
        pltpu.VMEM((CHUNK, CH), jnp.float32),
        pltpu.VMEM((CHUNK, CH), jnp.float32),
        pltpu.VMEM_SHARED((N_PAD, CH), jnp.float32),
        pltpu.SemaphoreType.DMA,
        pltpu.SemaphoreType.DMA,
    ],
)
def _sc_aggregate(g_hbm, edges_hbm, zeros_hbm, acc_out,
                  src_v, dst_v, rows_a, rows_b, shared_acc, sem_a, sem_b):
    c = lax.axis_index("c")
    s = lax.axis_index("s")
    wid = c * 16 + s

    pltpu.sync_copy(zeros_hbm, rows_a)
    for m in range(ROWS_PER_TILE // CHUNK):
        pltpu.sync_copy(rows_a,
                        shared_acc.at[pl.ds(s * ROWS_PER_TILE + m * CHUNK, CHUNK)])
    plsc.subcore_barrier()

    for grp in range(IDX_GROUPS):
        base = wid * EW_CHUNKS + grp * GCHUNKS
        pltpu.sync_copy(edges_hbm.at[pl.ds(base, GCHUNKS)], src_v)
        pltpu.sync_copy(edges_hbm.at[pl.ds(EROWS + base, GCHUNKS)], dst_v)
        pltpu.async_copy(g_hbm.at[src_v.at[0]], rows_a, sem_a)

        def body(j, carry):
            @pl.when(j % 2 == 0)
            def _():
                @pl.when(j + 1 < GCHUNKS)
                def _():
                    pltpu.async_copy(g_hbm.at[src_v.at[j + 1]], rows_b, sem_b)
                pltpu.make_async_copy(g_hbm.at[src_v.at[j]], rows_a, sem_a).wait()
                pltpu.sync_copy(rows_a, shared_acc.at[dst_v.at[j]], add=True)

            @pl.when(j % 2 == 1)
            def _():
                @pl.when(j + 1 < GCHUNKS)
                def _():
                    pltpu.async_copy(g_hbm.at[src_v.at[j + 1]], rows_a, sem_a)
                pltpu.make_async_copy(g_hbm.at[src_v.at[j]], rows_b, sem_b).wait()
                pltpu.sync_copy(rows_b, shared_acc.at[dst_v.at[j]], add=True)

            return carry

        lax.fori_loop(0, GCHUNKS, body, 0)
    plsc.subcore_barrier()

    out_sls = [pl.ds(s * ROWS_PER_TILE + m * CHUNK, CHUNK)
               for m in range(ROWS_PER_TILE // CHUNK)]
    bufs = [rows_a, rows_b]
    sems = [sem_a, sem_b]
    for m, sl in enumerate(out_sls):
        buf, sem = bufs[m % 2], sems[m % 2]
        if m >= 2:
            pltpu.make_async_copy(buf, acc_out.at[c].at[out_sls[m - 2]], sem).wait()
        pltpu.sync_copy(shared_acc.at[sl], buf)
        pltpu.async_copy(buf, acc_out.at[c].at[sl], sem)
    for m in (len(out_sls) - 2, len(out_sls) - 1):
        pltpu.make_async_copy(bufs[m % 2], acc_out.at[c].at[out_sls[m]],
                              sems[m % 2]).wait()


def _tc_prep(x, W, deg_col2):
    def body(x_ref, w_ref, deg_ref, g_ref, dfull_ref):
        deg = deg_ref[0, 0:N_NODES] + deg_ref[1, 0:N_NODES]
        dinv = lax.rsqrt(deg + 1.0)
        dfull = jnp.broadcast_to(dinv, (N_NODES, CH))
        h = jnp.dot(x_ref[...], w_ref[...], preferred_element_type=jnp.float32)
        g_ref[...] = h * dfull
        dfull_ref[...] = dfull

    return pl.pallas_call(
        body,
        out_shape=(
            jax.ShapeDtypeStruct((N_NODES, CH), jnp.float32),
            jax.ShapeDtypeStruct((N_NODES, CH), jnp.float32),
        ),
    )(x, W, deg_col2)


def _tc_combine(acc2, g, dfull, pos, W_pos, b_row, b_pos_row):
    def body(acc_ref, g_ref, dfull_ref, pos_ref, wp_ref, b_ref, bp_ref, out_ref):
        acc = acc_ref[0, 0:N_NODES] + acc_ref[1, 0:N_NODES] + g_ref[...]
        pos_lin = jnp.dot(pos_ref[...], wp_ref[...],
                          preferred_element_type=jnp.float32)
        out_ref[...] = dfull_ref[...] * acc + pos_lin + b_ref[...] + bp_ref[...]

    return pl.pallas_call(
        body,
        out_shape=jax.ShapeDtypeStruct((N_NODES, CH), jnp.float32),
    )(acc2, g, dfull, pos, W_pos, b_row, b_pos_row)


def kernel(x, edge_index, pos_encoding, W, b, W_pos, b_pos):
    n_extra = E_PAD - N_EDGES
    pad_iota = lax.iota(jnp.int32, n_extra)
    pad_vals = jnp.stack([pad_iota % N_NODES,
                          N_NODES + pad_iota % (N_PAD - N_NODES)])
    e2d = jnp.concatenate([edge_index.astype(jnp.int32), pad_vals],
                          axis=1).reshape(2 * EROWS, CHUNK)

    deg2 = _sc_degree(e2d)

    g, dfull = _tc_prep(x, W, deg2.reshape(2, N_PAD, 1))

    acc2 = _sc_aggregate(g, e2d, jnp.zeros((CHUNK, CH), jnp.float32))

    return _tc_combine(
        acc2,
        g,
        dfull,
        pos_encoding,
        W_pos,
        b.reshape(1, CH),
        b_pos.reshape(1, CH),
    )


# speedup vs baseline: 48.9835x; 1.1087x over previous
import functools

import jax
import jax.numpy as jnp
from jax import lax
from jax.experimental import pallas as pl
from jax.experimental.pallas import tpu as pltpu
from jax.experimental.pallas import tpu_sc as plsc

N_NODES = 10000
N_EDGES = 320000
CH = 128
POS_DIM = 16

NW = 32
CHUNK = 128
EROWS = N_EDGES // CHUNK
BASE_CHUNKS = EROWS // NW
EXTRA_WORKERS = EROWS - NW * BASE_CHUNKS
GCHUNKS = 40
N_PAD = 10240
ROWS_PER_TILE = N_PAD // 16

_mesh = plsc.VectorSubcoreMesh(core_axis_name="c", subcore_axis_name="s")


@functools.partial(
    pl.kernel,
    out_type=jax.ShapeDtypeStruct((2, N_PAD), jnp.float32),
    mesh=_mesh,
    scratch_types=[
        pltpu.VMEM((88, 128), jnp.int32),
        pltpu.VMEM((128,), jnp.float32),
        pltpu.VMEM((ROWS_PER_TILE,), jnp.float32),
        pltpu.VMEM_SHARED((N_PAD,), jnp.float32),
        pltpu.SemaphoreType.DMA,
    ],
)
def _sc_degree(edges_hbm, deg_out, idx_v, ones_v, stage_v, shared_deg, sem_d):
    c = lax.axis_index("c")
    s = lax.axis_index("s")
    wid = c * 16 + s
    t = jnp.where(wid < EXTRA_WORKERS, BASE_CHUNKS + 1, BASE_CHUNKS)
    base = BASE_CHUNKS * wid + jnp.minimum(wid, EXTRA_WORKERS)
    off_al = (base // 8) * 8
    start = base - off_al

    for i in range(128 // 16):
        ones_v[pl.ds(i * 16, 16)] = jnp.ones((16,), jnp.float32)
    for i in range(ROWS_PER_TILE // 16):
        stage_v[pl.ds(i * 16, 16)] = jnp.zeros((16,), jnp.float32)
    pltpu.sync_copy(stage_v, shared_deg.at[pl.ds(s * ROWS_PER_TILE, ROWS_PER_TILE)])
    pltpu.sync_copy(edges_hbm.at[1].at[pl.ds(off_al, 88)], idx_v)
    plsc.subcore_barrier()

    pltpu.async_copy(ones_v, shared_deg.at[idx_v.at[start]], sem_d, add=True)

    def body(j, carry):
        pltpu.async_copy(ones_v, shared_deg.at[idx_v.at[j]], sem_d, add=True)
        pltpu.make_async_copy(ones_v, shared_deg.at[idx_v.at[j - 1]],
                              sem_d).wait()
        return carry

    lax.fori_loop(start + 1, start + t, body, 0)
    pltpu.make_async_copy(ones_v, shared_deg.at[idx_v.at[start + t - 1]],
                          sem_d).wait()
    plsc.subcore_barrier()

    pltpu.sync_copy(shared_deg.at[pl.ds(s * ROWS_PER_TILE, ROWS_PER_TILE)], stage_v)
    pltpu.sync_copy(stage_v, deg_out.at[c].at[pl.ds(s * ROWS_PER_TILE, ROWS_PER_TILE)])


@functools.partial(
    pl.kernel,
    out_type=jax.ShapeDtypeStruct((2, N_PAD, CH), jnp.float32),
    mesh=_mesh,
    scratch_types=[
        pltpu.VMEM((48, CHUNK), jnp.int32),
        pltpu.VMEM((48, CHUNK), jnp.int32),
        pltpu.VMEM((CHUNK, CH), jnp.float32),
        pltpu.VMEM((CHUNK, CH), jnp.float32),
        pltpu.VMEM_SHARED((N_PAD, CH), jnp.float32),
        pltpu.SemaphoreType.DMA,
        pltpu.SemaphoreType.DMA,
    ],
)
def _sc_aggregate(g_hbm, edges_hbm, zeros_hbm, acc_out,
                  src_v, dst_v, rows_a, rows_b, shared_acc, sem_a, sem_b):
    c = lax.axis_index("c")
    s = lax.axis_index("s")
    wid = c * 16 + s

    pltpu.sync_copy(zeros_hbm, rows_a)
    for m in range(ROWS_PER_TILE // CHUNK):
        pltpu.sync_copy(rows_a,
                        shared_acc.at[pl.ds(s * ROWS_PER_TILE + m * CHUNK, CHUNK)])
    plsc.subcore_barrier()

    t = jnp.where(wid < EXTRA_WORKERS, BASE_CHUNKS + 1, BASE_CHUNKS)
    base = BASE_CHUNKS * wid + jnp.minimum(wid, EXTRA_WORKERS)
    for grp in range(2):
        lo = base if grp == 0 else base + GCHUNKS
        hi = base + GCHUNKS if grp == 0 else base + t
        off_al = (lo // 8) * 8
        start = lo - off_al
        end = hi - off_al
        pltpu.sync_copy(edges_hbm.at[0].at[pl.ds(off_al, 48)], src_v)
        pltpu.sync_copy(edges_hbm.at[1].at[pl.ds(off_al, 48)], dst_v)
        pltpu.async_copy(g_hbm.at[src_v.at[start]], rows_a, sem_a)

        def body(j, carry):
            k = j - start

            @pl.when(k % 2 == 0)
            def _():
                @pl.when(j + 1 < end)
                def _():
                    pltpu.async_copy(g_hbm.at[src_v.at[j + 1]], rows_b, sem_b)
                pltpu.make_async_copy(g_hbm.at[src_v.at[j]], rows_a, sem_a).wait()
                pltpu.sync_copy(rows_a, shared_acc.at[dst_v.at[j]], add=True)

            @pl.when(k % 2 == 1)
            def _():
                @pl.when(j + 1 < end)
                def _():
                    pltpu.async_copy(g_hbm.at[src_v.at[j + 1]], rows_a, sem_a)
                pltpu.make_async_copy(g_hbm.at[src_v.at[j]], rows_b, sem_b).wait()
                pltpu.sync_copy(rows_b, shared_acc.at[dst_v.at[j]], add=True)

            return carry

        lax.fori_loop(start, end, body, 0)
    plsc.subcore_barrier()

    out_sls = [pl.ds(s * ROWS_PER_TILE + m * CHUNK, CHUNK)
               for m in range(ROWS_PER_TILE // CHUNK)]
    bufs = [rows_a, rows_b]
    sems = [sem_a, sem_b]
    for m, sl in enumerate(out_sls):
        buf, sem = bufs[m % 2], sems[m % 2]
        if m >= 2:
            pltpu.make_async_copy(buf, acc_out.at[c].at[out_sls[m - 2]], sem).wait()
        pltpu.sync_copy(shared_acc.at[sl], buf)
        pltpu.async_copy(buf, acc_out.at[c].at[sl], sem)
    for m in (len(out_sls) - 2, len(out_sls) - 1):
        pltpu.make_async_copy(bufs[m % 2], acc_out.at[c].at[out_sls[m]],
                              sems[m % 2]).wait()


def _tc_matmul(x, W):
    def body(x_ref, w_ref, h_ref):
        h_ref[...] = jnp.dot(x_ref[...], w_ref[...],
                             preferred_element_type=jnp.float32)

    return pl.pallas_call(
        body,
        out_shape=jax.ShapeDtypeStruct((N_NODES, CH), jnp.float32),
    )(x, W)


def _tc_scale(h, deg2, pos8, W_big, b_row, b_pos_row):
    def body(h_ref, deg_ref, pos_ref, wp_ref, b_ref, bp_ref, g_ref, rest_ref):
        deg = deg_ref[0] + deg_ref[1]
        dinv = lax.rsqrt(deg + 1.0)
        dcol = dinv.reshape(N_PAD, 1)[0:N_NODES]
        dfull = jnp.broadcast_to(dcol, (N_NODES, CH))
        g = h_ref[...] * dfull
        pos_lin = jnp.dot(pos_ref[...], wp_ref[...],
                          preferred_element_type=jnp.float32)
        g_ref[...] = g
        rest_ref[...] = (g * dfull + pos_lin.reshape(N_NODES, CH)
                         + b_ref[...] + bp_ref[...])

    return pl.pallas_call(
        body,
        out_shape=(
            jax.ShapeDtypeStruct((N_NODES, CH), jnp.float32),
            jax.ShapeDtypeStruct((N_NODES, CH), jnp.float32),
        ),
    )(h, deg2, pos8, W_big, b_row, b_pos_row)


def _tc_combine(acc2, rest, deg2):
    def body(acc_ref, rest_ref, deg_ref, out_ref):
        deg = deg_ref[0] + deg_ref[1]
        dinv = lax.rsqrt(deg + 1.0)
        dcol = dinv.reshape(N_PAD, 1)[0:N_NODES]
        dfull = jnp.broadcast_to(dcol, (N_NODES, CH))
        acc = acc_ref[0, 0:N_NODES] + acc_ref[1, 0:N_NODES]
        out_ref[...] = dfull * acc + rest_ref[...]

    return pl.pallas_call(
        body,
        out_shape=jax.ShapeDtypeStruct((N_NODES, CH), jnp.float32),
    )(acc2, rest, deg2)


def kernel(x, edge_index, pos_encoding, W, b, W_pos, b_pos):
    e3d = jnp.pad(edge_index.astype(jnp.int32).reshape(2, EROWS, CHUNK),
                  ((0, 0), (0, 4), (0, 0)))

    eye8 = jnp.eye(8, dtype=W_pos.dtype)
    W_big = (eye8[:, None, :, None] * W_pos[None, :, None, :]).reshape(
        8 * POS_DIM, 8 * CH)

    h = _tc_matmul(x, W)
    deg2 = _sc_degree(e3d)
    g, rest = _tc_scale(h, deg2, pos_encoding.reshape(N_NODES // 8, 8 * POS_DIM),
                        W_big, b.reshape(1, CH), b_pos.reshape(1, CH))

    acc2 = _sc_aggregate(g, e3d, jnp.zeros((CHUNK, CH), jnp.float32))

    return _tc_combine(acc2, rest, deg2)
